# baseline (speedup 1.0000x reference)
import functools

import jax
import jax.numpy as jnp
from jax import lax
from jax.experimental import pallas as pl
from jax.experimental.pallas import tpu as pltpu
from jax.experimental.pallas import tpu_sc as plsc

NUM_EXPERTS = 64
TOP_K = 8
N_TOK = 32768
BT = 1024
BTT = 2048
NW = 32
TPW = N_TOK // NW
LANES = 16


def _logits_t_kernel(u_ref, w_ref, b_ref, o_ref):
    o_ref[...] = (
        lax.dot_general(
            w_ref[...], u_ref[...],
            (((1,), (1,)), ((), ())),
            preferred_element_type=jnp.float32,
        )
        + b_ref[...]
    )


def _sc_gate_body(lt_hbm, out_hbm, buf):
    tpw = buf.shape[1]
    wid = lax.axis_index("s") * 2 + lax.axis_index("c")
    base = wid * tpw
    pltpu.sync_copy(lt_hbm.at[:, pl.ds(base, tpw)], buf)

    def group(g, carry):
        cols = pl.ds(g * LANES, LANES)
        r = [jnp.full((LANES,), -jnp.inf, dtype=jnp.float32) for _ in range(TOP_K)]
        for e in range(NUM_EXPERTS):
            x = buf[e, cols]
            for i in range(TOP_K):
                hi = jnp.maximum(r[i], x)
                x = jnp.minimum(r[i], x)
                r[i] = hi
        thr = r[TOP_K - 1]
        mx = r[0]
        acc = jnp.zeros((LANES,), dtype=jnp.float32)
        for e in range(NUM_EXPERTS):
            x = buf[e, cols]
            ex = jnp.where(x >= thr, jnp.exp(x - mx), 0.0)
            buf[e, cols] = ex
            acc = acc + ex
        rinv = 1.0 / acc
        for e in range(NUM_EXPERTS):
            buf[e, cols] = buf[e, cols] * rinv
        return carry

    lax.fori_loop(0, tpw // LANES, group, 0)
    pltpu.sync_copy(buf, out_hbm.at[:, pl.ds(base, tpw)])


def _transpose_kernel(wt_ref, o_ref):
    o_ref[...] = wt_ref[...].T


NCHUNK = 4


def _chunk(u_c, W, b2):
    n_c = u_c.shape[0]
    logits_t = pl.pallas_call(
        _logits_t_kernel,
        grid=(n_c // BT,),
        in_specs=[
            pl.BlockSpec((BT, u_c.shape[1]), lambda i: (i, 0)),
            pl.BlockSpec((NUM_EXPERTS, W.shape[1]), lambda i: (0, 0)),
            pl.BlockSpec((NUM_EXPERTS, 1), lambda i: (0, 0)),
        ],
        out_specs=pl.BlockSpec((NUM_EXPERTS, BT), lambda i: (0, i)),
        out_shape=jax.ShapeDtypeStruct((NUM_EXPERTS, n_c), jnp.float32),
    )(u_c, W, b2)

    sc_gate = functools.partial(
        pl.kernel,
        mesh=plsc.VectorSubcoreMesh(core_axis_name="c", subcore_axis_name="s"),
        out_type=jax.ShapeDtypeStruct((NUM_EXPERTS, n_c), jnp.float32),
        scratch_types=[pltpu.VMEM((NUM_EXPERTS, n_c // NW), jnp.float32)],
    )(_sc_gate_body)
    w_t = sc_gate(logits_t)

    return pl.pallas_call(
        _transpose_kernel,
        grid=(n_c // BTT,),
        in_specs=[pl.BlockSpec((NUM_EXPERTS, BTT), lambda i: (0, i))],
        out_specs=pl.BlockSpec((BTT, NUM_EXPERTS), lambda i: (i, 0)),
        out_shape=jax.ShapeDtypeStruct((n_c, NUM_EXPERTS), jnp.float32),
    )(w_t)


@functools.partial(jax.jit, static_argnames=())
def kernel(h, u, W, b):
    del h
    n_tok = u.shape[0]
    b2 = b.reshape(NUM_EXPERTS, 1)
    step = n_tok // NCHUNK
    outs = [_chunk(u[i * step:(i + 1) * step], W, b2) for i in range(NCHUNK)]
    return jnp.concatenate(outs, axis=0)


# speedup vs baseline: 2.5036x; 2.5036x over previous
import functools

import jax
import jax.numpy as jnp
from jax import lax
from jax.experimental import pallas as pl
from jax.experimental.pallas import tpu as pltpu
from jax.experimental.pallas import tpu_sc as plsc

NUM_EXPERTS = 64
TOP_K = 8
N_TOK = 32768
BT = 1024
BTT = 2048
NW = 32
LANES = 16

_SORT8 = (
    (0, 1), (2, 3), (4, 5), (6, 7),
    (0, 2), (1, 3), (4, 6), (5, 7),
    (1, 2), (5, 6),
    (0, 4), (1, 5), (2, 6), (3, 7),
    (2, 4), (3, 5),
    (1, 2), (3, 4), (5, 6),
)
_BITONIC8 = (
    (0, 4), (1, 5), (2, 6), (3, 7),
    (0, 2), (1, 3), (4, 6), (5, 7),
    (0, 1), (2, 3), (4, 5), (6, 7),
)


def _ce(v, i, j):
    hi = jnp.maximum(v[i], v[j])
    v[j] = jnp.minimum(v[i], v[j])
    v[i] = hi


def _logits_t_kernel(u_ref, w_ref, b_ref, o_ref):
    o_ref[...] = (
        lax.dot_general(
            w_ref[...], u_ref[...],
            (((1,), (1,)), ((), ())),
            preferred_element_type=jnp.float32,
        )
        + b_ref[...]
    )


def _sc_gate_body(lt_hbm, out_hbm, buf):
    tpw = buf.shape[1]
    wid = lax.axis_index("s") * 2 + lax.axis_index("c")
    base = wid * tpw
    pltpu.sync_copy(lt_hbm.at[:, pl.ds(base, tpw)], buf)

    def group(g, carry):
        cols = pl.ds(g * LANES, LANES)
        top = None
        for blk in range(NUM_EXPERTS // TOP_K):
            v = [buf[blk * TOP_K + i, cols] for i in range(TOP_K)]
            for i, j in _SORT8:
                _ce(v, i, j)
            if top is None:
                top = v
            else:
                top = [jnp.maximum(top[i], v[TOP_K - 1 - i]) for i in range(TOP_K)]
                for i, j in _BITONIC8:
                    _ce(top, i, j)
        thr = top[TOP_K - 1]
        mx = top[0]
        acc = jnp.zeros((LANES,), dtype=jnp.float32)
        for e in range(NUM_EXPERTS):
            x = buf[e, cols]
            ex = jnp.where(x >= thr, jnp.exp(x - mx), 0.0)
            buf[e, cols] = ex
            acc = acc + ex
        rinv = 1.0 / acc
        for e in range(NUM_EXPERTS):
            buf[e, cols] = buf[e, cols] * rinv
        return carry

    lax.fori_loop(0, tpw // LANES, group, 0)
    pltpu.sync_copy(buf, out_hbm.at[:, pl.ds(base, tpw)])


def _transpose_kernel(wt_ref, o_ref):
    o_ref[...] = wt_ref[...].T


@functools.partial(jax.jit, static_argnames=())
def kernel(h, u, W, b):
    del h
    n_tok = u.shape[0]
    b2 = b.reshape(NUM_EXPERTS, 1)

    logits_t = pl.pallas_call(
        _logits_t_kernel,
        grid=(n_tok // BT,),
        in_specs=[
            pl.BlockSpec((BT, u.shape[1]), lambda i: (i, 0)),
            pl.BlockSpec((NUM_EXPERTS, W.shape[1]), lambda i: (0, 0)),
            pl.BlockSpec((NUM_EXPERTS, 1), lambda i: (0, 0)),
        ],
        out_specs=pl.BlockSpec((NUM_EXPERTS, BT), lambda i: (0, i)),
        out_shape=jax.ShapeDtypeStruct((NUM_EXPERTS, n_tok), jnp.float32),
    )(u, W, b2)

    sc_gate = functools.partial(
        pl.kernel,
        mesh=plsc.VectorSubcoreMesh(core_axis_name="c", subcore_axis_name="s"),
        out_type=jax.ShapeDtypeStruct((NUM_EXPERTS, n_tok), jnp.float32),
        scratch_types=[pltpu.VMEM((NUM_EXPERTS, n_tok // NW), jnp.float32)],
    )(_sc_gate_body)
    w_t = sc_gate(logits_t)

    return pl.pallas_call(
        _transpose_kernel,
        grid=(n_tok // BTT,),
        in_specs=[pl.BlockSpec((NUM_EXPERTS, BTT), lambda i: (0, i))],
        out_specs=pl.BlockSpec((BTT, NUM_EXPERTS), lambda i: (i, 0)),
        out_shape=jax.ShapeDtypeStruct((n_tok, NUM_EXPERTS), jnp.float32),
    )(w_t)


# speedup vs baseline: 2.5271x; 1.0094x over previous
import functools

import jax
import jax.numpy as jnp
from jax import lax
from jax.experimental import pallas as pl
from jax.experimental.pallas import tpu as pltpu
from jax.experimental.pallas import tpu_sc as plsc

NUM_EXPERTS = 64
TOP_K = 8
N_TOK = 32768
BT = 1024
BTT = 2048
NW = 32
LANES = 16

_SORT8 = (
    (0, 1), (2, 3), (4, 5), (6, 7),
    (0, 2), (1, 3), (4, 6), (5, 7),
    (1, 2), (5, 6),
    (0, 4), (1, 5), (2, 6), (3, 7),
    (2, 4), (3, 5),
    (1, 2), (3, 4), (5, 6),
)
_BITONIC8 = (
    (0, 4), (1, 5), (2, 6), (3, 7),
    (0, 2), (1, 3), (4, 6), (5, 7),
    (0, 1), (2, 3), (4, 5), (6, 7),
)


def _ce(v, i, j):
    hi = jnp.maximum(v[i], v[j])
    v[j] = jnp.minimum(v[i], v[j])
    v[i] = hi


def _logits_t_kernel(u_ref, w_ref, b_ref, o_ref):
    o_ref[...] = (
        lax.dot_general(
            w_ref[...], u_ref[...],
            (((1,), (1,)), ((), ())),
            preferred_element_type=jnp.float32,
        )
        + b_ref[...]
    )


def _sc_gate_body(lt_hbm, out_hbm, buf):
    tpw = buf.shape[1]
    wid = lax.axis_index("s") * 2 + lax.axis_index("c")
    base = wid * tpw
    pltpu.sync_copy(lt_hbm.at[:, pl.ds(base, tpw)], buf)

    def group(g, carry):
        cols = pl.ds(g * LANES, LANES)
        top = None
        for blk in range(NUM_EXPERTS // TOP_K):
            v = [buf[blk * TOP_K + i, cols] for i in range(TOP_K)]
            for i, j in _SORT8:
                _ce(v, i, j)
            if top is None:
                top = v
            else:
                top = [jnp.maximum(top[i], v[TOP_K - 1 - i]) for i in range(TOP_K)]
                for i, j in _BITONIC8:
                    _ce(top, i, j)
        thr = top[TOP_K - 1]
        mx = top[0]
        acc = jnp.full((LANES,), 1.0, dtype=jnp.float32)
        for i in range(1, TOP_K):
            acc = acc + jnp.exp(top[i] - mx)
        rinv = 1.0 / acc
        for e in range(NUM_EXPERTS):
            x = buf[e, cols]
            buf[e, cols] = jnp.where(x >= thr, jnp.exp(x - mx) * rinv, 0.0)
        return carry

    lax.fori_loop(0, tpw // LANES, group, 0)
    pltpu.sync_copy(buf, out_hbm.at[:, pl.ds(base, tpw)])


def _transpose_kernel(wt_ref, o_ref):
    o_ref[...] = wt_ref[...].T


@functools.partial(jax.jit, static_argnames=())
def kernel(h, u, W, b):
    del h
    n_tok = u.shape[0]
    b2 = b.reshape(NUM_EXPERTS, 1)

    logits_t = pl.pallas_call(
        _logits_t_kernel,
        grid=(n_tok // BT,),
        in_specs=[
            pl.BlockSpec((BT, u.shape[1]), lambda i: (i, 0)),
            pl.BlockSpec((NUM_EXPERTS, W.shape[1]), lambda i: (0, 0)),
            pl.BlockSpec((NUM_EXPERTS, 1), lambda i: (0, 0)),
        ],
        out_specs=pl.BlockSpec((NUM_EXPERTS, BT), lambda i: (0, i)),
        out_shape=jax.ShapeDtypeStruct((NUM_EXPERTS, n_tok), jnp.float32),
    )(u, W, b2)

    sc_gate = functools.partial(
        pl.kernel,
        mesh=plsc.VectorSubcoreMesh(core_axis_name="c", subcore_axis_name="s"),
        out_type=jax.ShapeDtypeStruct((NUM_EXPERTS, n_tok), jnp.float32),
        scratch_types=[pltpu.VMEM((NUM_EXPERTS, n_tok // NW), jnp.float32)],
    )(_sc_gate_body)
    w_t = sc_gate(logits_t)

    return pl.pallas_call(
        _transpose_kernel,
        grid=(n_tok // BTT,),
        in_specs=[pl.BlockSpec((NUM_EXPERTS, BTT), lambda i: (0, i))],
        out_specs=pl.BlockSpec((BTT, NUM_EXPERTS), lambda i: (i, 0)),
        out_shape=jax.ShapeDtypeStruct((n_tok, NUM_EXPERTS), jnp.float32),
    )(w_t)


# speedup vs baseline: 2.5375x; 1.0041x over previous
import functools

import jax
import jax.numpy as jnp
from jax import lax
from jax.experimental import pallas as pl
from jax.experimental.pallas import tpu as pltpu
from jax.experimental.pallas import tpu_sc as plsc

NUM_EXPERTS = 64
TOP_K = 8
N_TOK = 32768
BT = 1024
BTT = 2048
NW = 32
LANES = 16

_SORT8 = (
    (0, 1), (2, 3), (4, 5), (6, 7),
    (0, 2), (1, 3), (4, 6), (5, 7),
    (1, 2), (5, 6),
    (0, 4), (1, 5), (2, 6), (3, 7),
    (2, 4), (3, 5),
    (1, 2), (3, 4), (5, 6),
)
_BITONIC8 = (
    (0, 4), (1, 5), (2, 6), (3, 7),
    (0, 2), (1, 3), (4, 6), (5, 7),
    (0, 1), (2, 3), (4, 5), (6, 7),
)


def _ce(v, i, j):
    hi = jnp.maximum(v[i], v[j])
    v[j] = jnp.minimum(v[i], v[j])
    v[i] = hi


def _logits_t_kernel(u_ref, w_ref, b_ref, o_ref):
    o_ref[...] = (
        lax.dot_general(
            w_ref[...], u_ref[...],
            (((1,), (1,)), ((), ())),
            preferred_element_type=jnp.float32,
        )
        + b_ref[...]
    )


def _sc_gate_body(lt_hbm, out_hbm, buf_a, buf_b, sem_ia, sem_ib, sem_oa, sem_ob):
    half = buf_a.shape[1]
    tpw = 2 * half
    wid = lax.axis_index("s") * 2 + lax.axis_index("c")
    base = wid * tpw

    in_a = pltpu.async_copy(lt_hbm.at[:, pl.ds(base, half)], buf_a, sem_ia)
    in_b = pltpu.async_copy(lt_hbm.at[:, pl.ds(base + half, half)], buf_b, sem_ib)

    def group(g, buf):
        cols = pl.ds(g * LANES, LANES)
        top = None
        for blk in range(NUM_EXPERTS // TOP_K):
            v = [buf[blk * TOP_K + i, cols] for i in range(TOP_K)]
            for i, j in _SORT8:
                _ce(v, i, j)
            if top is None:
                top = v
            else:
                top = [jnp.maximum(top[i], v[TOP_K - 1 - i]) for i in range(TOP_K)]
                for i, j in _BITONIC8:
                    _ce(top, i, j)
        thr = top[TOP_K - 1]
        mx = top[0]
        acc = jnp.full((LANES,), 1.0, dtype=jnp.float32)
        for i in range(1, TOP_K):
            acc = acc + jnp.exp(top[i] - mx)
        rinv = 1.0 / acc
        for e in range(NUM_EXPERTS):
            x = buf[e, cols]
            buf[e, cols] = jnp.where(x >= thr, jnp.exp(x - mx) * rinv, 0.0)

    in_a.wait()
    lax.fori_loop(0, half // LANES, lambda g, c: (group(g, buf_a), c)[1], 0)
    out_a = pltpu.async_copy(buf_a, out_hbm.at[:, pl.ds(base, half)], sem_oa)
    in_b.wait()
    lax.fori_loop(0, half // LANES, lambda g, c: (group(g, buf_b), c)[1], 0)
    out_b = pltpu.async_copy(buf_b, out_hbm.at[:, pl.ds(base + half, half)], sem_ob)
    out_a.wait()
    out_b.wait()


def _transpose_kernel(wt_ref, o_ref):
    o_ref[...] = wt_ref[...].T


@functools.partial(jax.jit, static_argnames=())
def kernel(h, u, W, b):
    del h
    n_tok = u.shape[0]
    b2 = b.reshape(NUM_EXPERTS, 1)

    logits_t = pl.pallas_call(
        _logits_t_kernel,
        grid=(n_tok // BT,),
        in_specs=[
            pl.BlockSpec((BT, u.shape[1]), lambda i: (i, 0)),
            pl.BlockSpec((NUM_EXPERTS, W.shape[1]), lambda i: (0, 0)),
            pl.BlockSpec((NUM_EXPERTS, 1), lambda i: (0, 0)),
        ],
        out_specs=pl.BlockSpec((NUM_EXPERTS, BT), lambda i: (0, i)),
        out_shape=jax.ShapeDtypeStruct((NUM_EXPERTS, n_tok), jnp.float32),
    )(u, W, b2)

    sc_gate = functools.partial(
        pl.kernel,
        mesh=plsc.VectorSubcoreMesh(core_axis_name="c", subcore_axis_name="s"),
        out_type=jax.ShapeDtypeStruct((NUM_EXPERTS, n_tok), jnp.float32),
        scratch_types=[
            pltpu.VMEM((NUM_EXPERTS, n_tok // NW // 2), jnp.float32),
            pltpu.VMEM((NUM_EXPERTS, n_tok // NW // 2), jnp.float32),
            pltpu.SemaphoreType.DMA,
            pltpu.SemaphoreType.DMA,
            pltpu.SemaphoreType.DMA,
            pltpu.SemaphoreType.DMA,
        ],
    )(_sc_gate_body)
    w_t = sc_gate(logits_t)

    return pl.pallas_call(
        _transpose_kernel,
        grid=(n_tok // BTT,),
        in_specs=[pl.BlockSpec((NUM_EXPERTS, BTT), lambda i: (0, i))],
        out_specs=pl.BlockSpec((BTT, NUM_EXPERTS), lambda i: (i, 0)),
        out_shape=jax.ShapeDtypeStruct((n_tok, NUM_EXPERTS), jnp.float32),
    )(w_t)
